# baseline (speedup 1.0000x reference)
import functools

import jax
import jax.numpy as jnp
from jax import lax
from jax.experimental import pallas as pl
from jax.experimental.pallas import tpu as pltpu
from jax.experimental.pallas import tpu_sc as plsc

_NC = 2
_NS = 16
_NW = _NC * _NS

_CHUNK = 32
_NBUF = 2


@functools.partial(jax.jit, static_argnums=(2, 3))
def _sc_gather(idx, table, B, D):
    b_per_w = B // _NW
    n_chunks = b_per_w // _CHUNK
    mesh = plsc.VectorSubcoreMesh(core_axis_name="c", subcore_axis_name="s")

    @functools.partial(
        pl.kernel,
        out_type=jax.ShapeDtypeStruct((B, D), jnp.float32),
        mesh=mesh,
        scratch_types=[
            pltpu.VMEM((n_chunks, _CHUNK), jnp.int32),
            pltpu.VMEM((_NBUF, _CHUNK, D), jnp.float32),
            pltpu.SemaphoreType.DMA((_NBUF,)),
        ],
    )
    def k(idx_hbm, table_hbm, out_hbm, idx_v, rows_v, gsem):
        wid = lax.axis_index("s") * _NC + lax.axis_index("c")
        base = wid * b_per_w
        pltpu.sync_copy(idx_hbm.at[wid], idx_v)

        def gather(j, b):
            pltpu.async_copy(table_hbm.at[idx_v.at[j]], rows_v.at[b],
                             gsem.at[b])

        def wait_gather(j, b):
            pltpu.make_async_copy(table_hbm.at[idx_v.at[j]], rows_v.at[b],
                                  gsem.at[b]).wait()

        for b in range(_NBUF):
            gather(b, b)

        @pl.loop(0, n_chunks, step=_NBUF)
        def _(j0):
            for b in range(_NBUF):
                j = j0 + b
                wait_gather(j, b)
                pltpu.sync_copy(rows_v.at[b],
                                out_hbm.at[pl.ds(base + j * _CHUNK, _CHUNK)])

                @pl.when(j + _NBUF < n_chunks)
                def _():
                    gather(j + _NBUF, b)

    return k(idx, table)


def kernel(x, embed_positions):
    BATCH, SEQ = x.shape
    V, D = embed_positions.shape
    B = BATCH * SEQ
    b_per_w = B // _NW
    idx = x.astype(jnp.int32).reshape(_NW, b_per_w // _CHUNK, _CHUNK)
    out = _sc_gather(idx, embed_positions, B, D)
    return out.reshape(BATCH, SEQ, D)


# speedup vs baseline: 1.0036x; 1.0036x over previous
import functools

import jax
import jax.numpy as jnp
from jax import lax
from jax.experimental import pallas as pl
from jax.experimental.pallas import tpu as pltpu
from jax.experimental.pallas import tpu_sc as plsc

_NC = 2
_NS = 16
_NW = _NC * _NS

_CHUNK = 32
_NBUF = 2


@functools.partial(jax.jit, static_argnums=(2, 3))
def _sc_gather(idx, table, B, D):
    b_per_w = B // _NW
    n_chunks = b_per_w // _CHUNK
    mesh = plsc.VectorSubcoreMesh(core_axis_name="c", subcore_axis_name="s")

    @functools.partial(
        pl.kernel,
        out_type=jax.ShapeDtypeStruct((B, D), jnp.float32),
        mesh=mesh,
        scratch_types=[
            pltpu.VMEM((b_per_w,), jnp.int32),
            pltpu.VMEM((_NBUF, _CHUNK, D), jnp.float32),
            pltpu.SemaphoreType.DMA((_NBUF,)),
        ],
    )
    def k(idx_hbm, table_hbm, out_hbm, idx_v, rows_v, gsem):
        wid = lax.axis_index("s") * _NC + lax.axis_index("c")
        base = wid * b_per_w
        w_per_row = idx_hbm.shape[1] // b_per_w
        pltpu.sync_copy(
            idx_hbm.at[wid // w_per_row,
                       pl.ds((wid % w_per_row) * b_per_w, b_per_w)],
            idx_v)

        def gather(j, b):
            pltpu.async_copy(
                table_hbm.at[idx_v.at[pl.ds(j * _CHUNK, _CHUNK)]],
                rows_v.at[b], gsem.at[b])

        def wait_gather(j, b):
            pltpu.make_async_copy(
                table_hbm.at[idx_v.at[pl.ds(j * _CHUNK, _CHUNK)]],
                rows_v.at[b], gsem.at[b]).wait()

        for b in range(_NBUF):
            gather(b, b)

        @pl.loop(0, n_chunks, step=_NBUF)
        def _(j0):
            for b in range(_NBUF):
                j = j0 + b
                wait_gather(j, b)
                pltpu.sync_copy(rows_v.at[b],
                                out_hbm.at[pl.ds(base + j * _CHUNK, _CHUNK)])

                @pl.when(j + _NBUF < n_chunks)
                def _():
                    gather(j + _NBUF, b)

    return k(idx, table)


def kernel(x, embed_positions):
    BATCH, SEQ = x.shape
    V, D = embed_positions.shape
    B = BATCH * SEQ
    out = _sc_gather(x.astype(jnp.int32), embed_positions, B, D)
    return out.reshape(BATCH, SEQ, D)
